# baseline (device time: 80789 ns/iter reference)
import jax
import jax.numpy as jnp
from jax import lax
from jax.experimental import pallas as pl
from jax.experimental.pallas import tpu as pltpu

M = 2048
PAD = 8
M2 = M + PAD
NBITS = 9


def _body(scal_ref, xs_ref, out_ref, recv_ref, send_sems, recv_sems):
    my_xi = lax.axis_index("x")
    my_yi = lax.axis_index("y")
    peer = (1 - my_xi, my_yi)

    k8 = scal_ref[0]
    k = scal_ref[1]
    n_keep = scal_ref[2]
    m8 = k8 * 8

    barrier = pltpu.get_barrier_semaphore()
    pl.semaphore_signal(
        barrier, inc=1, device_id=peer, device_id_type=pl.DeviceIdType.MESH
    )
    pl.semaphore_wait(barrier, 1)

    def rdma_for_bit(b):
        rows = 8 << b
        off = ((k8 >> (b + 1)) << (b + 1)) * 8
        return pltpu.make_async_remote_copy(
            src_ref=xs_ref.at[pl.ds(off, rows), :],
            dst_ref=recv_ref.at[pl.ds(off, rows), :],
            send_sem=send_sems.at[b],
            recv_sem=recv_sems.at[b],
            device_id=peer,
            device_id_type=pl.DeviceIdType.MESH,
        )

    for b in range(NBITS - 1, -1, -1):
        @pl.when(((k8 >> b) & 1) == 1)
        def _(b=b):
            rdma_for_bit(b).start()

    for b in range(NBITS - 1, -1, -1):
        @pl.when(((k8 >> b) & 1) == 1)
        def _(b=b):
            rdma_for_bit(b).wait()

    dev0 = my_xi == 0
    shift_keep = jnp.where(dev0, (M2 - m8) % M2, M2 + k - m8)
    shift_recv = jnp.where(dev0, n_keep, 0)
    rolled_keep = pltpu.roll(xs_ref[:, :], shift_keep % M2, axis=0)
    rolled_recv = pltpu.roll(recv_ref[:, :], shift_recv, axis=0)
    rows_iota = lax.broadcasted_iota(jnp.int32, (M, 1), 0)
    keep_lo = jnp.where(dev0, 0, k)
    sel_keep = (rows_iota >= keep_lo) & (rows_iota < keep_lo + n_keep)
    out_ref[:, :] = jnp.where(sel_keep, rolled_keep[:M], rolled_recv[:M])


def kernel(x, dest):
    m, n = x.shape
    my_xi = lax.axis_index("x")

    xb = x.astype(jnp.bfloat16)
    to_peer = (dest != my_xi).astype(jnp.int32)
    k = jnp.sum(to_peer)
    k8 = (k + 7) >> 3
    m8 = k8 * 8

    excl_peer = jnp.cumsum(to_peer) - to_peer
    keep = 1 - to_peer
    excl_keep = jnp.cumsum(keep) - keep
    d = jnp.where(to_peer == 1, excl_peer, m8 + excl_keep)
    g = jnp.zeros((M2,), jnp.int32).at[d].set(jnp.arange(m, dtype=jnp.int32))
    xs = xb[g]

    scal = jnp.stack([k8, k, m - k]).astype(jnp.int32)

    return pl.pallas_call(
        _body,
        out_shape=jax.ShapeDtypeStruct((m, n), jnp.bfloat16),
        in_specs=[
            pl.BlockSpec(memory_space=pltpu.SMEM),
            pl.BlockSpec(memory_space=pltpu.VMEM),
        ],
        out_specs=pl.BlockSpec(memory_space=pltpu.VMEM),
        scratch_shapes=[
            pltpu.VMEM((M2, n), jnp.bfloat16),
            pltpu.SemaphoreType.DMA((NBITS,)),
            pltpu.SemaphoreType.DMA((NBITS,)),
        ],
        compiler_params=pltpu.CompilerParams(
            collective_id=0, vmem_limit_bytes=100 * 1024 * 1024
        ),
    )(scal, xs)
